# baseline (speedup 1.0000x reference)
import functools

import jax
import jax.numpy as jnp
from jax import lax
from jax.experimental import pallas as pl
from jax.experimental.pallas import tpu as pltpu
from jax.experimental.pallas import tpu_sc as plsc

N = 10000
E = 320000
D = 128
H = 128
C = 40
CP = 128
EPS = 1e-5

NC = 2
NS = 16
NW = NC * NS
CHUNK = 96
STEPS = 105
EDGES_PER_TILE = STEPS * CHUNK
EPAD = NW * EDGES_PER_TILE
NBUF = 2
STRIPE = 640
NP = NS * STRIPE
ZR = 128

_MESH = plsc.VectorSubcoreMesh(core_axis_name="c", subcore_axis_name="s")


def _zero_fill(buf, rows, width):
    def body(i, carry):
        r = i // (width // 16)
        q = i % (width // 16)
        buf[r, pl.ds(q * 16, 16)] = jnp.zeros((16,), jnp.float32)
        return carry

    lax.fori_loop(0, rows * (width // 16), body, 0)


@functools.partial(
    pl.kernel,
    out_type=jax.ShapeDtypeStruct((NC, NP, 16), jnp.float32),
    mesh=_MESH,
    scratch_types=[
        pltpu.VMEM((STEPS, CHUNK), jnp.int32),
        pltpu.VMEM((CHUNK, 16), jnp.float32),
        pltpu.VMEM((8, 16), jnp.float32),
        pltpu.VMEM_SHARED((NP, 16), jnp.float32),
    ],
)
def _deg_kernel(dst_hbm, out_hbm, di_v, ones_v, zrow_v, acc_sh):
    c = lax.axis_index("c")
    s = lax.axis_index("s")
    wid = s * NC + c

    pltpu.sync_copy(dst_hbm.at[wid], di_v)

    def fill_ones(i, carry):
        ones_v[i, :] = jnp.ones((16,), jnp.float32)
        return carry

    lax.fori_loop(0, CHUNK, fill_ones, 0)
    _zero_fill(zrow_v, 8, 16)

    def zstep(z, carry):
        pltpu.sync_copy(zrow_v, acc_sh.at[pl.ds(s * STRIPE + z * 8, 8)])
        return carry

    lax.fori_loop(0, STRIPE // 8, zstep, 0)
    plsc.subcore_barrier()

    def step(j, carry):
        pltpu.sync_copy(ones_v, acc_sh.at[di_v.at[j]], add=True)
        return carry

    lax.fori_loop(0, STEPS, step, 0)
    plsc.subcore_barrier()
    pltpu.sync_copy(
        acc_sh.at[pl.ds(s * STRIPE, STRIPE)],
        out_hbm.at[c, pl.ds(s * STRIPE, STRIPE)],
    )


def _make_msg_kernel(width):
    @functools.partial(
        pl.kernel,
        out_type=jax.ShapeDtypeStruct((NC, NP, width), jnp.float32),
        mesh=_MESH,
        scratch_types=[
            pltpu.VMEM((EDGES_PER_TILE,), jnp.int32),
            pltpu.VMEM((STEPS, CHUNK), jnp.int32),
            pltpu.VMEM((NBUF, CHUNK, width), jnp.float32),
            pltpu.VMEM_SHARED((NP, width), jnp.float32),
            [pltpu.SemaphoreType.DMA] * NBUF,
        ],
    )
    def _msg(y_hbm, src_hbm, dst_hbm, out_hbm, si_v, di_v, rows_v, acc_sh, sems):
        c = lax.axis_index("c")
        s = lax.axis_index("s")
        wid = s * NC + c

        pltpu.sync_copy(src_hbm.at[wid], si_v)
        pltpu.sync_copy(dst_hbm.at[wid], di_v)
        _zero_fill(rows_v.at[0], CHUNK, width)

        def zstep(z, carry):
            pltpu.sync_copy(
                rows_v.at[0], acc_sh.at[pl.ds(s * STRIPE + z * CHUNK, CHUNK)]
            )
            return carry

        lax.fori_loop(0, STRIPE // CHUNK, zstep, 0)
        plsc.subcore_barrier()

        for b in range(NBUF):
            pltpu.async_copy(
                y_hbm.at[si_v.at[pl.ds(b * CHUNK, CHUNK)]], rows_v.at[b], sems[b]
            )

        def step(jj, carry):
            for u in range(NBUF):
                j = jj * NBUF + u

                @pl.when(j < STEPS)
                def _():
                    pltpu.make_async_copy(
                        y_hbm.at[si_v.at[pl.ds(j * CHUNK, CHUNK)]],
                        rows_v.at[u],
                        sems[u],
                    ).wait()
                    pltpu.sync_copy(
                        rows_v.at[u], acc_sh.at[di_v.at[j]], add=True
                    )

                    @pl.when(j + NBUF < STEPS)
                    def _():
                        pltpu.async_copy(
                            y_hbm.at[si_v.at[pl.ds((j + NBUF) * CHUNK, CHUNK)]],
                            rows_v.at[u],
                            sems[u],
                        )

            return carry

        lax.fori_loop(0, (STEPS + NBUF - 1) // NBUF, step, 0)
        plsc.subcore_barrier()
        pltpu.sync_copy(
            acc_sh.at[pl.ds(s * STRIPE, STRIPE)],
            out_hbm.at[c, pl.ds(s * STRIPE, STRIPE)],
        )

    return _msg


_msg_h = _make_msg_kernel(H)

NB = 2000


def _tca_body(degp_ref, x_ref, w_ref, dinv_ref, y_ref):
    deg = degp_ref[0, :, 0:1] + degp_ref[1, :, 0:1] + 1.0
    dinv = lax.rsqrt(deg)
    dinv_ref[...] = dinv
    y_ref[...] = (
        jnp.dot(x_ref[...], w_ref[...], preferred_element_type=jnp.float32) * dinv
    )


def _tca(degp, x, w):
    return pl.pallas_call(
        _tca_body,
        grid=(N // NB,),
        in_specs=[
            pl.BlockSpec((2, NB, 16), lambda i: (0, i, 0)),
            pl.BlockSpec((NB, D), lambda i: (i, 0)),
            pl.BlockSpec((D, H), lambda i: (0, 0)),
        ],
        out_specs=[
            pl.BlockSpec((NB, 1), lambda i: (i, 0)),
            pl.BlockSpec((NB, H), lambda i: (i, 0)),
        ],
        out_shape=[
            jax.ShapeDtypeStruct((N, 1), jnp.float32),
            jax.ShapeDtypeStruct((N, H), jnp.float32),
        ],
    )(degp, x, w)


def _tcb_body(accp_ref, y_ref, dinv_ref, st_ref, w_ref, out_ref):
    dinv = dinv_ref[...]
    z = (accp_ref[0] + accp_ref[1] + y_ref[...]) * dinv
    h = jnp.maximum(z * st_ref[0:1, :] + st_ref[1:2, :], 0.0)
    out_ref[...] = (
        jnp.dot(h, w_ref[...], preferred_element_type=jnp.float32) * dinv
    )


def _tcb(accp, y, dinv, st, w):
    wo = w.shape[1]
    return pl.pallas_call(
        _tcb_body,
        grid=(N // NB,),
        in_specs=[
            pl.BlockSpec((2, NB, H), lambda i: (0, i, 0)),
            pl.BlockSpec((NB, H), lambda i: (i, 0)),
            pl.BlockSpec((NB, 1), lambda i: (i, 0)),
            pl.BlockSpec((2, H), lambda i: (0, 0)),
            pl.BlockSpec((H, wo), lambda i: (0, 0)),
        ],
        out_specs=pl.BlockSpec((NB, wo), lambda i: (i, 0)),
        out_shape=jax.ShapeDtypeStruct((N, wo), jnp.float32),
    )(accp, y, dinv, st, w)


def _tcc_body(accp_ref, y_ref, dinv_ref, b_ref, out_ref):
    out_ref[...] = (
        accp_ref[0] + accp_ref[1] + y_ref[...]
    ) * dinv_ref[...] + b_ref[0:1, :]


def _tcc(accp, y, dinv, b):
    return pl.pallas_call(
        _tcc_body,
        grid=(N // NB,),
        in_specs=[
            pl.BlockSpec((2, NB, CP), lambda i: (0, i, 0)),
            pl.BlockSpec((NB, CP), lambda i: (i, 0)),
            pl.BlockSpec((NB, 1), lambda i: (i, 0)),
            pl.BlockSpec((1, CP), lambda i: (0, 0)),
        ],
        out_specs=pl.BlockSpec((NB, CP), lambda i: (i, 0)),
        out_shape=jax.ShapeDtypeStruct((N, CP), jnp.float32),
    )(accp, y, dinv, b)


def kernel(x, edge_index, W1, b1, W2, b2, W3, b3, g1, beta1, rm1, rv1, g2, beta2, rm2, rv2):
    src = jnp.pad(edge_index[0], (0, EPAD - E)).reshape(NW, EDGES_PER_TILE)
    pad_dst = N + (jnp.arange(EPAD - E, dtype=jnp.int32) % (NP - N))
    dst = jnp.concatenate([edge_index[1], pad_dst]).reshape(NW, STEPS, CHUNK)
    s1 = g1 * lax.rsqrt(rv1 + EPS)
    t1 = (b1 - rm1) * s1 + beta1
    s2 = g2 * lax.rsqrt(rv2 + EPS)
    t2 = (b2 - rm2) * s2 + beta2
    st1 = jnp.stack([s1, t1])
    st2 = jnp.stack([s2, t2])
    w3p = jnp.pad(W3, ((0, 0), (0, CP - C)))
    b3p = jnp.pad(b3, (0, CP - C)).reshape(1, CP)

    degp = _deg_kernel(dst)
    dinv, y1 = _tca(degp, x, W1)
    acc1 = _msg_h(y1, src, dst)
    y2 = _tcb(acc1, y1, dinv, st1, W2)
    acc2 = _msg_h(y2, src, dst)
    y3 = _tcb(acc2, y2, dinv, st2, w3p)
    acc3 = _msg_h(y3, src, dst)
    outp = _tcc(acc3, y3, dinv, b3p)
    return outp[:, :C]


# speedup vs baseline: 1.7289x; 1.7289x over previous
import functools

import jax
import jax.numpy as jnp
from jax import lax
from jax.experimental import pallas as pl
from jax.experimental.pallas import tpu as pltpu
from jax.experimental.pallas import tpu_sc as plsc

N = 10000
E = 320000
D = 128
H = 128
C = 40
CP = 128
EPS = 1e-5

NC = 2
NS = 16
NW = NC * NS
CHUNK = 80
EDGES_PER_TILE = E // NW
STEPS = EDGES_PER_TILE // CHUNK
NBUF = 2
STRIPE = 640
NP = NS * STRIPE
ZR = 128

_MESH = plsc.VectorSubcoreMesh(core_axis_name="c", subcore_axis_name="s")


@functools.partial(
    pl.kernel,
    out_type=jax.ShapeDtypeStruct((NC, NP, 16), jnp.float32),
    mesh=_MESH,
    scratch_types=[
        pltpu.VMEM((STEPS, CHUNK), jnp.int32),
        pltpu.VMEM((CHUNK, 16), jnp.float32),
        pltpu.VMEM((8, 16), jnp.float32),
        pltpu.VMEM_SHARED((NP, 16), jnp.float32),
    ],
)
def _deg_kernel(dst_hbm, onesz_hbm, out_hbm, di_v, ones_v, zrow_v, acc_sh):
    c = lax.axis_index("c")
    s = lax.axis_index("s")
    wid = s * NC + c

    pltpu.sync_copy(dst_hbm.at[wid], di_v)
    pltpu.sync_copy(onesz_hbm.at[pl.ds(0, CHUNK)], ones_v)
    pltpu.sync_copy(onesz_hbm.at[pl.ds(CHUNK, 8)], zrow_v)

    def zstep(z, carry):
        pltpu.sync_copy(zrow_v, acc_sh.at[pl.ds(s * STRIPE + z * 8, 8)])
        return carry

    lax.fori_loop(0, STRIPE // 8, zstep, 0)
    plsc.subcore_barrier()

    def step(j, carry):
        pltpu.sync_copy(ones_v, acc_sh.at[di_v.at[j]], add=True)
        return carry

    lax.fori_loop(0, STEPS, step, 0)
    plsc.subcore_barrier()
    pltpu.sync_copy(
        acc_sh.at[pl.ds(s * STRIPE, STRIPE)],
        out_hbm.at[c, pl.ds(s * STRIPE, STRIPE)],
    )


def _make_msg_kernel(width):
    @functools.partial(
        pl.kernel,
        out_type=jax.ShapeDtypeStruct((NC, NP, width), jnp.float32),
        mesh=_MESH,
        scratch_types=[
            pltpu.VMEM((EDGES_PER_TILE,), jnp.int32),
            pltpu.VMEM((STEPS, CHUNK), jnp.int32),
            pltpu.VMEM((NBUF, CHUNK, width), jnp.float32),
            pltpu.VMEM_SHARED((NP, width), jnp.float32),
            [pltpu.SemaphoreType.DMA] * NBUF,
        ],
    )
    def _msg(y_hbm, src_hbm, dst_hbm, zeros_hbm, out_hbm, si_v, di_v, rows_v, acc_sh, sems):
        c = lax.axis_index("c")
        s = lax.axis_index("s")
        wid = s * NC + c

        pltpu.sync_copy(src_hbm.at[wid], si_v)
        pltpu.sync_copy(dst_hbm.at[wid], di_v)
        pltpu.sync_copy(zeros_hbm, rows_v.at[0])

        def zstep(z, carry):
            pltpu.sync_copy(
                rows_v.at[0], acc_sh.at[pl.ds(s * STRIPE + z * CHUNK, CHUNK)]
            )
            return carry

        lax.fori_loop(0, STRIPE // CHUNK, zstep, 0)
        plsc.subcore_barrier()

        for b in range(NBUF):
            pltpu.async_copy(
                y_hbm.at[si_v.at[pl.ds(b * CHUNK, CHUNK)]], rows_v.at[b], sems[b]
            )

        def step(jj, carry):
            for u in range(NBUF):
                j = jj * NBUF + u

                @pl.when(j < STEPS)
                def _():
                    pltpu.make_async_copy(
                        y_hbm.at[si_v.at[pl.ds(j * CHUNK, CHUNK)]],
                        rows_v.at[u],
                        sems[u],
                    ).wait()
                    pltpu.sync_copy(
                        rows_v.at[u], acc_sh.at[di_v.at[j]], add=True
                    )

                    @pl.when(j + NBUF < STEPS)
                    def _():
                        pltpu.async_copy(
                            y_hbm.at[si_v.at[pl.ds((j + NBUF) * CHUNK, CHUNK)]],
                            rows_v.at[u],
                            sems[u],
                        )

            return carry

        lax.fori_loop(0, (STEPS + NBUF - 1) // NBUF, step, 0)
        plsc.subcore_barrier()
        pltpu.sync_copy(
            acc_sh.at[pl.ds(s * STRIPE, STRIPE)],
            out_hbm.at[c, pl.ds(s * STRIPE, STRIPE)],
        )

    return _msg


_msg_h = _make_msg_kernel(H)

NB = 2000


def _tca_body(degp_ref, x_ref, w_ref, dinv_ref, y_ref):
    deg = degp_ref[0, :, 0:1] + degp_ref[1, :, 0:1] + 1.0
    dinv = lax.rsqrt(deg)
    dinv_ref[...] = dinv
    y_ref[...] = (
        jnp.dot(x_ref[...], w_ref[...], preferred_element_type=jnp.float32) * dinv
    )


def _tca(degp, x, w):
    return pl.pallas_call(
        _tca_body,
        grid=(N // NB,),
        in_specs=[
            pl.BlockSpec((2, NB, 16), lambda i: (0, i, 0)),
            pl.BlockSpec((NB, D), lambda i: (i, 0)),
            pl.BlockSpec((D, H), lambda i: (0, 0)),
        ],
        out_specs=[
            pl.BlockSpec((NB, 1), lambda i: (i, 0)),
            pl.BlockSpec((NB, H), lambda i: (i, 0)),
        ],
        out_shape=[
            jax.ShapeDtypeStruct((N, 1), jnp.float32),
            jax.ShapeDtypeStruct((N, H), jnp.float32),
        ],
    )(degp, x, w)


def _tcb_body(accp_ref, y_ref, dinv_ref, st_ref, w_ref, out_ref):
    dinv = dinv_ref[...]
    z = (accp_ref[0] + accp_ref[1] + y_ref[...]) * dinv
    h = jnp.maximum(z * st_ref[0:1, :] + st_ref[1:2, :], 0.0)
    out_ref[...] = (
        jnp.dot(h, w_ref[...], preferred_element_type=jnp.float32) * dinv
    )


def _tcb(accp, y, dinv, st, w):
    wo = w.shape[1]
    return pl.pallas_call(
        _tcb_body,
        grid=(N // NB,),
        in_specs=[
            pl.BlockSpec((2, NB, H), lambda i: (0, i, 0)),
            pl.BlockSpec((NB, H), lambda i: (i, 0)),
            pl.BlockSpec((NB, 1), lambda i: (i, 0)),
            pl.BlockSpec((2, H), lambda i: (0, 0)),
            pl.BlockSpec((H, wo), lambda i: (0, 0)),
        ],
        out_specs=pl.BlockSpec((NB, wo), lambda i: (i, 0)),
        out_shape=jax.ShapeDtypeStruct((N, wo), jnp.float32),
    )(accp, y, dinv, st, w)


def _tcc_body(accp_ref, y_ref, dinv_ref, b_ref, out_ref):
    out_ref[...] = (
        accp_ref[0] + accp_ref[1] + y_ref[...]
    ) * dinv_ref[...] + b_ref[0:1, :]


def _tcc(accp, y, dinv, b):
    return pl.pallas_call(
        _tcc_body,
        grid=(N // NB,),
        in_specs=[
            pl.BlockSpec((2, NB, CP), lambda i: (0, i, 0)),
            pl.BlockSpec((NB, CP), lambda i: (i, 0)),
            pl.BlockSpec((NB, 1), lambda i: (i, 0)),
            pl.BlockSpec((1, CP), lambda i: (0, 0)),
        ],
        out_specs=pl.BlockSpec((NB, CP), lambda i: (i, 0)),
        out_shape=jax.ShapeDtypeStruct((N, CP), jnp.float32),
    )(accp, y, dinv, b)


def kernel(x, edge_index, W1, b1, W2, b2, W3, b3, g1, beta1, rm1, rv1, g2, beta2, rm2, rv2):
    src = edge_index[0].reshape(NW, EDGES_PER_TILE)
    dst = edge_index[1].reshape(NW, STEPS, CHUNK)
    s1 = g1 * lax.rsqrt(rv1 + EPS)
    t1 = (b1 - rm1) * s1 + beta1
    s2 = g2 * lax.rsqrt(rv2 + EPS)
    t2 = (b2 - rm2) * s2 + beta2
    st1 = jnp.stack([s1, t1])
    st2 = jnp.stack([s2, t2])
    w3p = jnp.pad(W3, ((0, 0), (0, CP - C)))
    b3p = jnp.pad(b3, (0, CP - C)).reshape(1, CP)

    onesz = jnp.concatenate(
        [jnp.ones((CHUNK, 16), jnp.float32), jnp.zeros((8, 16), jnp.float32)]
    )
    zeros_h = jnp.zeros((CHUNK, H), jnp.float32)
    degp = _deg_kernel(dst, onesz)
    dinv, y1 = _tca(degp, x, W1)
    acc1 = _msg_h(y1, src, dst, zeros_h)
    y2 = _tcb(acc1, y1, dinv, st1, W2)
    acc2 = _msg_h(y2, src, dst, zeros_h)
    y3 = _tcb(acc2, y2, dinv, st2, w3p)
    acc3 = _msg_h(y3, src, dst, zeros_h)
    outp = _tcc(acc3, y3, dinv, b3p)
    return outp[:, :C]


# speedup vs baseline: 1.8170x; 1.0510x over previous
import functools

import jax
import jax.numpy as jnp
from jax import lax
from jax.experimental import pallas as pl
from jax.experimental.pallas import tpu as pltpu
from jax.experimental.pallas import tpu_sc as plsc

N = 10000
E = 320000
D = 128
H = 128
C = 40
CP = 64
EPS = 1e-5

NC = 2
NS = 16
NW = NC * NS
CHUNK = 80
EDGES_PER_TILE = E // NW
STEPS = EDGES_PER_TILE // CHUNK
NBUF = 2
STRIPE = 640
NP = NS * STRIPE
ZR = 128

_MESH = plsc.VectorSubcoreMesh(core_axis_name="c", subcore_axis_name="s")


@functools.partial(
    pl.kernel,
    out_type=jax.ShapeDtypeStruct((NC, NP, 16), jnp.float32),
    mesh=_MESH,
    scratch_types=[
        pltpu.VMEM((STEPS, CHUNK), jnp.int32),
        pltpu.VMEM((CHUNK, 16), jnp.float32),
        pltpu.VMEM((8, 16), jnp.float32),
        pltpu.VMEM_SHARED((NP, 16), jnp.float32),
    ],
)
def _deg_kernel(dst_hbm, onesz_hbm, out_hbm, di_v, ones_v, zrow_v, acc_sh):
    c = lax.axis_index("c")
    s = lax.axis_index("s")
    wid = s * NC + c

    pltpu.sync_copy(dst_hbm.at[wid], di_v)
    pltpu.sync_copy(onesz_hbm.at[pl.ds(0, CHUNK)], ones_v)
    pltpu.sync_copy(onesz_hbm.at[pl.ds(CHUNK, 8)], zrow_v)

    def zstep(z, carry):
        pltpu.sync_copy(zrow_v, acc_sh.at[pl.ds(s * STRIPE + z * 8, 8)])
        return carry

    lax.fori_loop(0, STRIPE // 8, zstep, 0)
    plsc.subcore_barrier()

    def step(j, carry):
        pltpu.sync_copy(ones_v, acc_sh.at[di_v.at[j]], add=True)
        return carry

    lax.fori_loop(0, STEPS, step, 0)
    plsc.subcore_barrier()
    pltpu.sync_copy(
        acc_sh.at[pl.ds(s * STRIPE, STRIPE)],
        out_hbm.at[c, pl.ds(s * STRIPE, STRIPE)],
    )


def _make_msg_kernel(width, tc_tiling=True):
    @functools.partial(
        pl.kernel,
        out_type=jax.ShapeDtypeStruct((NC, NP, width), jnp.float32),
        mesh=_MESH,
        compiler_params=pltpu.CompilerParams(use_tc_tiling_on_sc=tc_tiling),
        scratch_types=[
            pltpu.VMEM((EDGES_PER_TILE,), jnp.int32),
            pltpu.VMEM((STEPS, CHUNK), jnp.int32),
            pltpu.VMEM((NBUF, CHUNK, width), jnp.float32),
            pltpu.VMEM_SHARED((NP, width), jnp.float32),
            [pltpu.SemaphoreType.DMA] * NBUF,
        ],
    )
    def _msg(y_hbm, src_hbm, dst_hbm, zeros_hbm, out_hbm, si_v, di_v, rows_v, acc_sh, sems):
        c = lax.axis_index("c")
        s = lax.axis_index("s")
        wid = s * NC + c

        pltpu.sync_copy(src_hbm.at[wid], si_v)
        pltpu.sync_copy(dst_hbm.at[wid], di_v)
        pltpu.sync_copy(zeros_hbm, rows_v.at[0])

        def zstep(z, carry):
            pltpu.sync_copy(
                rows_v.at[0], acc_sh.at[pl.ds(s * STRIPE + z * CHUNK, CHUNK)]
            )
            return carry

        lax.fori_loop(0, STRIPE // CHUNK, zstep, 0)
        plsc.subcore_barrier()

        for b in range(NBUF):
            pltpu.async_copy(
                y_hbm.at[si_v.at[pl.ds(b * CHUNK, CHUNK)]], rows_v.at[b], sems[b]
            )

        def step(jj, carry):
            for u in range(NBUF):
                j = jj * NBUF + u

                @pl.when(j < STEPS)
                def _():
                    pltpu.make_async_copy(
                        y_hbm.at[si_v.at[pl.ds(j * CHUNK, CHUNK)]],
                        rows_v.at[u],
                        sems[u],
                    ).wait()
                    pltpu.sync_copy(
                        rows_v.at[u], acc_sh.at[di_v.at[j]], add=True
                    )

                    @pl.when(j + NBUF < STEPS)
                    def _():
                        pltpu.async_copy(
                            y_hbm.at[si_v.at[pl.ds((j + NBUF) * CHUNK, CHUNK)]],
                            rows_v.at[u],
                            sems[u],
                        )

            return carry

        lax.fori_loop(0, (STEPS + NBUF - 1) // NBUF, step, 0)
        plsc.subcore_barrier()
        pltpu.sync_copy(
            acc_sh.at[pl.ds(s * STRIPE, STRIPE)],
            out_hbm.at[c, pl.ds(s * STRIPE, STRIPE)],
        )

    return _msg


_msg_h = _make_msg_kernel(H)
_msg_c = _make_msg_kernel(CP, tc_tiling=False)

NB = 2000


def _tca_body(degp_ref, x_ref, w_ref, dinv_ref, y_ref):
    deg = degp_ref[0, :, 0:1] + degp_ref[1, :, 0:1] + 1.0
    dinv = lax.rsqrt(deg)
    dinv_ref[...] = dinv
    y_ref[...] = (
        jnp.dot(x_ref[...], w_ref[...], preferred_element_type=jnp.float32) * dinv
    )


def _tca(degp, x, w):
    return pl.pallas_call(
        _tca_body,
        grid=(N // NB,),
        in_specs=[
            pl.BlockSpec((2, NB, 16), lambda i: (0, i, 0)),
            pl.BlockSpec((NB, D), lambda i: (i, 0)),
            pl.BlockSpec((D, H), lambda i: (0, 0)),
        ],
        out_specs=[
            pl.BlockSpec((NB, 1), lambda i: (i, 0)),
            pl.BlockSpec((NB, H), lambda i: (i, 0)),
        ],
        out_shape=[
            jax.ShapeDtypeStruct((N, 1), jnp.float32),
            jax.ShapeDtypeStruct((N, H), jnp.float32),
        ],
    )(degp, x, w)


def _tcb_body(accp_ref, y_ref, dinv_ref, st_ref, w_ref, out_ref):
    dinv = dinv_ref[...]
    z = (accp_ref[0] + accp_ref[1] + y_ref[...]) * dinv
    h = jnp.maximum(z * st_ref[0:1, :] + st_ref[1:2, :], 0.0)
    out_ref[...] = (
        jnp.dot(h, w_ref[...], preferred_element_type=jnp.float32) * dinv
    )


def _tcb(accp, y, dinv, st, w):
    wo = w.shape[1]
    return pl.pallas_call(
        _tcb_body,
        grid=(N // NB,),
        in_specs=[
            pl.BlockSpec((2, NB, H), lambda i: (0, i, 0)),
            pl.BlockSpec((NB, H), lambda i: (i, 0)),
            pl.BlockSpec((NB, 1), lambda i: (i, 0)),
            pl.BlockSpec((2, H), lambda i: (0, 0)),
            pl.BlockSpec((H, wo), lambda i: (0, 0)),
        ],
        out_specs=pl.BlockSpec((NB, wo), lambda i: (i, 0)),
        out_shape=jax.ShapeDtypeStruct((N, wo), jnp.float32),
    )(accp, y, dinv, st, w)


def _tcc_body(accp_ref, y_ref, dinv_ref, b_ref, out_ref):
    out_ref[...] = (
        accp_ref[0] + accp_ref[1] + y_ref[...]
    ) * dinv_ref[...] + b_ref[0:1, :]


def _tcc(accp, y, dinv, b):
    return pl.pallas_call(
        _tcc_body,
        grid=(N // NB,),
        in_specs=[
            pl.BlockSpec((2, NB, CP), lambda i: (0, i, 0)),
            pl.BlockSpec((NB, CP), lambda i: (i, 0)),
            pl.BlockSpec((NB, 1), lambda i: (i, 0)),
            pl.BlockSpec((1, CP), lambda i: (0, 0)),
        ],
        out_specs=pl.BlockSpec((NB, CP), lambda i: (i, 0)),
        out_shape=jax.ShapeDtypeStruct((N, CP), jnp.float32),
    )(accp, y, dinv, b)


def kernel(x, edge_index, W1, b1, W2, b2, W3, b3, g1, beta1, rm1, rv1, g2, beta2, rm2, rv2):
    src = edge_index[0].reshape(NW, EDGES_PER_TILE)
    dst = edge_index[1].reshape(NW, STEPS, CHUNK)
    s1 = g1 * lax.rsqrt(rv1 + EPS)
    t1 = (b1 - rm1) * s1 + beta1
    s2 = g2 * lax.rsqrt(rv2 + EPS)
    t2 = (b2 - rm2) * s2 + beta2
    st1 = jnp.stack([s1, t1])
    st2 = jnp.stack([s2, t2])
    w3p = jnp.pad(W3, ((0, 0), (0, CP - C)))
    b3p = jnp.pad(b3, (0, CP - C)).reshape(1, CP)

    onesz = jnp.concatenate(
        [jnp.ones((CHUNK, 16), jnp.float32), jnp.zeros((8, 16), jnp.float32)]
    )
    zeros_h = jnp.zeros((CHUNK, H), jnp.float32)
    zeros_c = jnp.zeros((CHUNK, CP), jnp.float32)
    degp = _deg_kernel(dst, onesz)
    dinv, y1 = _tca(degp, x, W1)
    acc1 = _msg_h(y1, src, dst, zeros_h)
    y2 = _tcb(acc1, y1, dinv, st1, W2)
    acc2 = _msg_h(y2, src, dst, zeros_h)
    y3 = _tcb(acc2, y2, dinv, st2, w3p)
    acc3 = _msg_c(y3, src, dst, zeros_c)
    outp = _tcc(acc3, y3, dinv, b3p)
    return outp[:, :C]


# speedup vs baseline: 2.0540x; 1.1304x over previous
import functools

import jax
import jax.numpy as jnp
from jax import lax
from jax.experimental import pallas as pl
from jax.experimental.pallas import tpu as pltpu
from jax.experimental.pallas import tpu_sc as plsc

N = 10000
E = 320000
D = 128
H = 128
C = 40
CP = 64
EPS = 1e-5

NC = 2
NS = 16
NW = NC * NS
CHUNK = 80
EDGES_PER_TILE = E // NW
STEPS = EDGES_PER_TILE // CHUNK
NBUF = 3
STRIPE = 640
NP = NS * STRIPE
ZR = 128

_MESH = plsc.VectorSubcoreMesh(core_axis_name="c", subcore_axis_name="s")


@functools.partial(
    pl.kernel,
    out_type=jax.ShapeDtypeStruct((NC, NP, 16), jnp.float32),
    mesh=_MESH,
    scratch_types=[
        pltpu.VMEM((STEPS, CHUNK), jnp.int32),
        pltpu.VMEM((CHUNK, 16), jnp.float32),
        pltpu.VMEM((8, 16), jnp.float32),
        pltpu.VMEM_SHARED((NP, 16), jnp.float32),
    ],
)
def _deg_kernel(dst_hbm, onesz_hbm, out_hbm, di_v, ones_v, zrow_v, acc_sh):
    c = lax.axis_index("c")
    s = lax.axis_index("s")
    wid = s * NC + c

    pltpu.sync_copy(dst_hbm.at[wid], di_v)
    pltpu.sync_copy(onesz_hbm.at[pl.ds(0, CHUNK)], ones_v)
    pltpu.sync_copy(onesz_hbm.at[pl.ds(CHUNK, 8)], zrow_v)

    def zstep(z, carry):
        pltpu.sync_copy(zrow_v, acc_sh.at[pl.ds(s * STRIPE + z * 8, 8)])
        return carry

    lax.fori_loop(0, STRIPE // 8, zstep, 0)
    plsc.subcore_barrier()

    def step(j, carry):
        pltpu.sync_copy(ones_v, acc_sh.at[di_v.at[j]], add=True)
        return carry

    lax.fori_loop(0, STEPS, step, 0)
    plsc.subcore_barrier()
    pltpu.sync_copy(
        acc_sh.at[pl.ds(s * STRIPE, STRIPE)],
        out_hbm.at[c, pl.ds(s * STRIPE, STRIPE)],
    )


def _make_msg_kernel(width, tc_tiling=True):
    rpt = N // NS

    @functools.partial(
        pl.kernel,
        out_type=jax.ShapeDtypeStruct((NC, N, width), jnp.float32),
        mesh=_MESH,
        compiler_params=pltpu.CompilerParams(use_tc_tiling_on_sc=False),
        scratch_types=[
            pltpu.VMEM((EDGES_PER_TILE,), jnp.int32),
            pltpu.VMEM((STEPS, CHUNK), jnp.int32),
            pltpu.VMEM((NBUF, CHUNK, width), jnp.float32),
            pltpu.VMEM_SHARED((N, width), jnp.float32),
            [pltpu.SemaphoreType.DMA] * (2 * NBUF),
        ],
    )
    def _msg(y_hbm, src_hbm, dst_hbm, zeros_hbm, out_hbm, si_v, di_v, rows_v, acc_sh, sems):
        c = lax.axis_index("c")
        s = lax.axis_index("s")
        wid = s * NC + c
        gsem = sems[:NBUF]
        ssem = sems[NBUF:]

        pltpu.sync_copy(src_hbm.at[wid], si_v)
        pltpu.sync_copy(dst_hbm.at[wid], di_v)
        pltpu.sync_copy(zeros_hbm, rows_v.at[0])
        for z in range(rpt // CHUNK):
            pltpu.sync_copy(
                rows_v.at[0], acc_sh.at[pl.ds(s * rpt + z * CHUNK, CHUNK)]
            )
        tail = rpt % CHUNK
        pltpu.sync_copy(
            rows_v.at[0].at[pl.ds(0, tail)],
            acc_sh.at[pl.ds(s * rpt + rpt - tail, tail)],
        )
        plsc.subcore_barrier()

        for b in range(NBUF - 1):
            pltpu.async_copy(
                y_hbm.at[si_v.at[pl.ds(b * CHUNK, CHUNK)]], rows_v.at[b], gsem[b]
            )

        def step(jj, carry):
            for u in range(NBUF):
                j = jj * NBUF + u

                @pl.when(j < STEPS)
                def _():
                    pltpu.make_async_copy(
                        y_hbm.at[si_v.at[pl.ds(j * CHUNK, CHUNK)]],
                        rows_v.at[u],
                        gsem[u],
                    ).wait()
                    pltpu.async_copy(
                        rows_v.at[u], acc_sh.at[di_v.at[j]], ssem[u], add=True
                    )

                    @pl.when(j == 0)
                    def _():
                        pltpu.async_copy(
                            y_hbm.at[si_v.at[pl.ds(2 * CHUNK, CHUNK)]],
                            rows_v.at[2],
                            gsem[2],
                        )

                    @pl.when((j >= 1) & (j + NBUF - 1 < STEPS))
                    def _():
                        up = (u + NBUF - 1) % NBUF
                        pltpu.make_async_copy(
                            rows_v.at[up], acc_sh.at[di_v.at[j - 1]], ssem[up]
                        ).wait()
                        pltpu.async_copy(
                            y_hbm.at[si_v.at[pl.ds((j + NBUF - 1) * CHUNK, CHUNK)]],
                            rows_v.at[up],
                            gsem[up],
                        )

            return carry

        lax.fori_loop(0, (STEPS + NBUF - 1) // NBUF, step, 0)
        for k in range(NBUF):
            j = STEPS - NBUF + k
            pltpu.make_async_copy(
                rows_v.at[j % NBUF], acc_sh.at[di_v.at[j]], ssem[j % NBUF]
            ).wait()
        plsc.subcore_barrier()
        pltpu.sync_copy(
            acc_sh.at[pl.ds(s * rpt, rpt)],
            out_hbm.at[c, pl.ds(s * rpt, rpt)],
        )

    return _msg


_msg_h = _make_msg_kernel(H)
_msg_c = _make_msg_kernel(CP, tc_tiling=False)

NB = 2000


def _tca_body(degp_ref, x_ref, w_ref, dinv_ref, y_ref):
    deg = degp_ref[0, :, 0:1] + degp_ref[1, :, 0:1] + 1.0
    dinv = lax.rsqrt(deg)
    dinv_ref[...] = dinv
    y_ref[...] = (
        jnp.dot(x_ref[...], w_ref[...], preferred_element_type=jnp.float32) * dinv
    )


def _tca(degp, x, w):
    return pl.pallas_call(
        _tca_body,
        grid=(N // NB,),
        in_specs=[
            pl.BlockSpec((2, NB, 16), lambda i: (0, i, 0)),
            pl.BlockSpec((NB, D), lambda i: (i, 0)),
            pl.BlockSpec((D, H), lambda i: (0, 0)),
        ],
        out_specs=[
            pl.BlockSpec((NB, 1), lambda i: (i, 0)),
            pl.BlockSpec((NB, H), lambda i: (i, 0)),
        ],
        out_shape=[
            jax.ShapeDtypeStruct((N, 1), jnp.float32),
            jax.ShapeDtypeStruct((N, H), jnp.float32),
        ],
    )(degp, x, w)


def _tcb_body(accp_ref, y_ref, dinv_ref, st_ref, w_ref, out_ref):
    dinv = dinv_ref[...]
    z = (accp_ref[0] + accp_ref[1] + y_ref[...]) * dinv
    h = jnp.maximum(z * st_ref[0:1, :] + st_ref[1:2, :], 0.0)
    out_ref[...] = (
        jnp.dot(h, w_ref[...], preferred_element_type=jnp.float32) * dinv
    )


def _tcb(accp, y, dinv, st, w):
    wo = w.shape[1]
    return pl.pallas_call(
        _tcb_body,
        grid=(N // NB,),
        in_specs=[
            pl.BlockSpec((2, NB, H), lambda i: (0, i, 0)),
            pl.BlockSpec((NB, H), lambda i: (i, 0)),
            pl.BlockSpec((NB, 1), lambda i: (i, 0)),
            pl.BlockSpec((2, H), lambda i: (0, 0)),
            pl.BlockSpec((H, wo), lambda i: (0, 0)),
        ],
        out_specs=pl.BlockSpec((NB, wo), lambda i: (i, 0)),
        out_shape=jax.ShapeDtypeStruct((N, wo), jnp.float32),
    )(accp, y, dinv, st, w)


def _tcc_body(accp_ref, y_ref, dinv_ref, b_ref, out_ref):
    out_ref[...] = (
        accp_ref[0] + accp_ref[1] + y_ref[...]
    ) * dinv_ref[...] + b_ref[0:1, :]


def _tcc(accp, y, dinv, b):
    return pl.pallas_call(
        _tcc_body,
        grid=(N // NB,),
        in_specs=[
            pl.BlockSpec((2, NB, CP), lambda i: (0, i, 0)),
            pl.BlockSpec((NB, CP), lambda i: (i, 0)),
            pl.BlockSpec((NB, 1), lambda i: (i, 0)),
            pl.BlockSpec((1, CP), lambda i: (0, 0)),
        ],
        out_specs=pl.BlockSpec((NB, CP), lambda i: (i, 0)),
        out_shape=jax.ShapeDtypeStruct((N, CP), jnp.float32),
    )(accp, y, dinv, b)


def kernel(x, edge_index, W1, b1, W2, b2, W3, b3, g1, beta1, rm1, rv1, g2, beta2, rm2, rv2):
    src = edge_index[0].reshape(NW, EDGES_PER_TILE)
    dst = edge_index[1].reshape(NW, STEPS, CHUNK)
    s1 = g1 * lax.rsqrt(rv1 + EPS)
    t1 = (b1 - rm1) * s1 + beta1
    s2 = g2 * lax.rsqrt(rv2 + EPS)
    t2 = (b2 - rm2) * s2 + beta2
    st1 = jnp.stack([s1, t1])
    st2 = jnp.stack([s2, t2])
    w3p = jnp.pad(W3, ((0, 0), (0, CP - C)))
    b3p = jnp.pad(b3, (0, CP - C)).reshape(1, CP)

    onesz = jnp.concatenate(
        [jnp.ones((CHUNK, 16), jnp.float32), jnp.zeros((8, 16), jnp.float32)]
    )
    zeros_h = jnp.zeros((CHUNK, H), jnp.float32)
    zeros_c = jnp.zeros((CHUNK, CP), jnp.float32)
    degp = _deg_kernel(dst, onesz)
    dinv, y1 = _tca(degp, x, W1)
    acc1 = _msg_h(y1, src, dst, zeros_h)
    y2 = _tcb(acc1, y1, dinv, st1, W2)
    acc2 = _msg_h(y2, src, dst, zeros_h)
    y3 = _tcb(acc2, y2, dinv, st2, w3p)
    acc3 = _msg_c(y3, src, dst, zeros_c)
    outp = _tcc(acc3, y3, dinv, b3p)
    return outp[:, :C]


# speedup vs baseline: 2.0908x; 1.0179x over previous
import functools

import jax
import jax.numpy as jnp
from jax import lax
from jax.experimental import pallas as pl
from jax.experimental.pallas import tpu as pltpu
from jax.experimental.pallas import tpu_sc as plsc

N = 10000
E = 320000
D = 128
H = 128
C = 40
CP = 64
EPS = 1e-5

NC = 2
NS = 16
NW = NC * NS
CHUNK = 80
EDGES_PER_TILE = E // NW
STEPS = EDGES_PER_TILE // CHUNK
NBUF = 3
STRIPE = 640
NP = NS * STRIPE
ZR = 128

_MESH = plsc.VectorSubcoreMesh(core_axis_name="c", subcore_axis_name="s")


@functools.partial(
    pl.kernel,
    out_type=jax.ShapeDtypeStruct((NC, NP, 16), jnp.float32),
    mesh=_MESH,
    scratch_types=[
        pltpu.VMEM((STEPS, CHUNK), jnp.int32),
        pltpu.VMEM((CHUNK, 16), jnp.float32),
        pltpu.VMEM((8, 16), jnp.float32),
        pltpu.VMEM_SHARED((NP, 16), jnp.float32),
        pltpu.SemaphoreType.DMA,
    ],
)
def _deg_kernel(dst_hbm, onesz_hbm, out_hbm, di_v, ones_v, zrow_v, acc_sh, dsem):
    c = lax.axis_index("c")
    s = lax.axis_index("s")
    wid = s * NC + c

    pltpu.sync_copy(dst_hbm.at[wid], di_v)
    pltpu.sync_copy(onesz_hbm.at[pl.ds(0, CHUNK)], ones_v)
    pltpu.sync_copy(onesz_hbm.at[pl.ds(CHUNK, 8)], zrow_v)

    def zstep(z, carry):
        pltpu.sync_copy(zrow_v, acc_sh.at[pl.ds(s * STRIPE + z * 8, 8)])
        return carry

    lax.fori_loop(0, STRIPE // 8, zstep, 0)
    plsc.subcore_barrier()

    def step(j, carry):
        pltpu.async_copy(ones_v, acc_sh.at[di_v.at[j]], dsem, add=True)
        return carry

    lax.fori_loop(0, STEPS, step, 0)

    def drain(j, carry):
        pltpu.make_async_copy(ones_v, acc_sh.at[di_v.at[j]], dsem).wait()
        return carry

    lax.fori_loop(0, STEPS, drain, 0)
    plsc.subcore_barrier()
    pltpu.sync_copy(
        acc_sh.at[pl.ds(s * STRIPE, STRIPE)],
        out_hbm.at[c, pl.ds(s * STRIPE, STRIPE)],
    )


def _make_msg_kernel(width, tc_tiling=True):
    rpt = N // NS

    @functools.partial(
        pl.kernel,
        out_type=jax.ShapeDtypeStruct((NC, N, width), jnp.float32),
        mesh=_MESH,
        compiler_params=pltpu.CompilerParams(use_tc_tiling_on_sc=False),
        scratch_types=[
            pltpu.VMEM((EDGES_PER_TILE,), jnp.int32),
            pltpu.VMEM((STEPS, CHUNK), jnp.int32),
            pltpu.VMEM((NBUF, CHUNK, width), jnp.float32),
            pltpu.VMEM_SHARED((N, width), jnp.float32),
            [pltpu.SemaphoreType.DMA] * (2 * NBUF),
        ],
    )
    def _msg(y_hbm, src_hbm, dst_hbm, zeros_hbm, out_hbm, si_v, di_v, rows_v, acc_sh, sems):
        c = lax.axis_index("c")
        s = lax.axis_index("s")
        wid = s * NC + c
        gsem = sems[:NBUF]
        ssem = sems[NBUF:]

        pltpu.sync_copy(src_hbm.at[wid], si_v)
        pltpu.sync_copy(dst_hbm.at[wid], di_v)
        pltpu.sync_copy(zeros_hbm, rows_v.at[0])
        for z in range(rpt // CHUNK):
            pltpu.sync_copy(
                rows_v.at[0], acc_sh.at[pl.ds(s * rpt + z * CHUNK, CHUNK)]
            )
        tail = rpt % CHUNK
        pltpu.sync_copy(
            rows_v.at[0].at[pl.ds(0, tail)],
            acc_sh.at[pl.ds(s * rpt + rpt - tail, tail)],
        )
        plsc.subcore_barrier()

        for b in range(NBUF - 1):
            pltpu.async_copy(
                y_hbm.at[si_v.at[pl.ds(b * CHUNK, CHUNK)]], rows_v.at[b], gsem[b]
            )

        def step(jj, carry):
            for u in range(NBUF):
                j = jj * NBUF + u

                @pl.when(j < STEPS)
                def _():
                    pltpu.make_async_copy(
                        y_hbm.at[si_v.at[pl.ds(j * CHUNK, CHUNK)]],
                        rows_v.at[u],
                        gsem[u],
                    ).wait()
                    pltpu.async_copy(
                        rows_v.at[u], acc_sh.at[di_v.at[j]], ssem[u], add=True
                    )

                    @pl.when(j == 0)
                    def _():
                        pltpu.async_copy(
                            y_hbm.at[si_v.at[pl.ds(2 * CHUNK, CHUNK)]],
                            rows_v.at[2],
                            gsem[2],
                        )

                    @pl.when((j >= 1) & (j + NBUF - 1 < STEPS))
                    def _():
                        up = (u + NBUF - 1) % NBUF
                        pltpu.make_async_copy(
                            rows_v.at[up], acc_sh.at[di_v.at[j - 1]], ssem[up]
                        ).wait()
                        pltpu.async_copy(
                            y_hbm.at[si_v.at[pl.ds((j + NBUF - 1) * CHUNK, CHUNK)]],
                            rows_v.at[up],
                            gsem[up],
                        )

            return carry

        lax.fori_loop(0, (STEPS + NBUF - 1) // NBUF, step, 0)
        for k in range(NBUF):
            j = STEPS - NBUF + k
            pltpu.make_async_copy(
                rows_v.at[j % NBUF], acc_sh.at[di_v.at[j]], ssem[j % NBUF]
            ).wait()
        plsc.subcore_barrier()
        pltpu.sync_copy(
            acc_sh.at[pl.ds(s * rpt, rpt)],
            out_hbm.at[c, pl.ds(s * rpt, rpt)],
        )

    return _msg


_msg_h = _make_msg_kernel(H)
_msg_c = _make_msg_kernel(CP, tc_tiling=False)

NB = 2000


def _tca_body(degp_ref, x_ref, w_ref, dinv_ref, y_ref):
    deg = degp_ref[0, :, 0:1] + degp_ref[1, :, 0:1] + 1.0
    dinv = lax.rsqrt(deg)
    dinv_ref[...] = dinv
    y_ref[...] = (
        jnp.dot(x_ref[...], w_ref[...], preferred_element_type=jnp.float32) * dinv
    )


def _tca(degp, x, w):
    return pl.pallas_call(
        _tca_body,
        grid=(N // NB,),
        in_specs=[
            pl.BlockSpec((2, NB, 16), lambda i: (0, i, 0)),
            pl.BlockSpec((NB, D), lambda i: (i, 0)),
            pl.BlockSpec((D, H), lambda i: (0, 0)),
        ],
        out_specs=[
            pl.BlockSpec((NB, 1), lambda i: (i, 0)),
            pl.BlockSpec((NB, H), lambda i: (i, 0)),
        ],
        out_shape=[
            jax.ShapeDtypeStruct((N, 1), jnp.float32),
            jax.ShapeDtypeStruct((N, H), jnp.float32),
        ],
    )(degp, x, w)


def _tcb_body(accp_ref, y_ref, dinv_ref, st_ref, w_ref, out_ref):
    dinv = dinv_ref[...]
    z = (accp_ref[0] + accp_ref[1] + y_ref[...]) * dinv
    h = jnp.maximum(z * st_ref[0:1, :] + st_ref[1:2, :], 0.0)
    out_ref[...] = (
        jnp.dot(h, w_ref[...], preferred_element_type=jnp.float32) * dinv
    )


def _tcb(accp, y, dinv, st, w):
    wo = w.shape[1]
    return pl.pallas_call(
        _tcb_body,
        grid=(N // NB,),
        in_specs=[
            pl.BlockSpec((2, NB, H), lambda i: (0, i, 0)),
            pl.BlockSpec((NB, H), lambda i: (i, 0)),
            pl.BlockSpec((NB, 1), lambda i: (i, 0)),
            pl.BlockSpec((2, H), lambda i: (0, 0)),
            pl.BlockSpec((H, wo), lambda i: (0, 0)),
        ],
        out_specs=pl.BlockSpec((NB, wo), lambda i: (i, 0)),
        out_shape=jax.ShapeDtypeStruct((N, wo), jnp.float32),
    )(accp, y, dinv, st, w)


def _tcc_body(accp_ref, y_ref, dinv_ref, b_ref, out_ref):
    out_ref[...] = (
        accp_ref[0] + accp_ref[1] + y_ref[...]
    ) * dinv_ref[...] + b_ref[0:1, :]


def _tcc(accp, y, dinv, b):
    return pl.pallas_call(
        _tcc_body,
        grid=(N // NB,),
        in_specs=[
            pl.BlockSpec((2, NB, CP), lambda i: (0, i, 0)),
            pl.BlockSpec((NB, CP), lambda i: (i, 0)),
            pl.BlockSpec((NB, 1), lambda i: (i, 0)),
            pl.BlockSpec((1, CP), lambda i: (0, 0)),
        ],
        out_specs=pl.BlockSpec((NB, CP), lambda i: (i, 0)),
        out_shape=jax.ShapeDtypeStruct((N, CP), jnp.float32),
    )(accp, y, dinv, b)


def kernel(x, edge_index, W1, b1, W2, b2, W3, b3, g1, beta1, rm1, rv1, g2, beta2, rm2, rv2):
    src = edge_index[0].reshape(NW, EDGES_PER_TILE)
    dst = edge_index[1].reshape(NW, STEPS, CHUNK)
    s1 = g1 * lax.rsqrt(rv1 + EPS)
    t1 = (b1 - rm1) * s1 + beta1
    s2 = g2 * lax.rsqrt(rv2 + EPS)
    t2 = (b2 - rm2) * s2 + beta2
    st1 = jnp.stack([s1, t1])
    st2 = jnp.stack([s2, t2])
    w3p = jnp.pad(W3, ((0, 0), (0, CP - C)))
    b3p = jnp.pad(b3, (0, CP - C)).reshape(1, CP)

    onesz = jnp.concatenate(
        [jnp.ones((CHUNK, 16), jnp.float32), jnp.zeros((8, 16), jnp.float32)]
    )
    zeros_h = jnp.zeros((CHUNK, H), jnp.float32)
    zeros_c = jnp.zeros((CHUNK, CP), jnp.float32)
    degp = _deg_kernel(dst, onesz)
    dinv, y1 = _tca(degp, x, W1)
    acc1 = _msg_h(y1, src, dst, zeros_h)
    y2 = _tcb(acc1, y1, dinv, st1, W2)
    acc2 = _msg_h(y2, src, dst, zeros_h)
    y3 = _tcb(acc2, y2, dinv, st2, w3p)
    acc3 = _msg_c(y3, src, dst, zeros_c)
    outp = _tcc(acc3, y3, dinv, b3p)
    return outp[:, :C]
